# baseline (speedup 1.0000x reference)
import functools

import jax
import jax.numpy as jnp
from jax import lax
from jax.experimental import pallas as pl
from jax.experimental.pallas import tpu as pltpu
from jax.experimental.pallas import tpu_sc as plsc

N_NODES = 10000
N_EDGES = 160000
D_FEAT = 256

_ROWS_PER_BLK = 2048

_info = plsc.get_sparse_core_info()
_NC, _NS, _L = _info.num_cores, _info.num_subcores, _info.num_lanes
_NW = _NC * _NS
_EPW = 4992
_XB = 128
_EPW_PAD = _EPW + _XB


_D_ROWS = (N_NODES + _ROWS_PER_BLK - 1) // _ROWS_PER_BLK
_D_SHIFT = 11
_D_MASK = _ROWS_PER_BLK - 1


def _matvec_body(x_ref, wt_ref, b_ref, o_ref):
    wv = wt_ref[0:1, :] - wt_ref[1:2, :]
    db = (b_ref[0, 0] - b_ref[0, 1]) * 0.5
    p = lax.dot_general(
        wv, x_ref[...], (((1,), (1,)), ((), ())),
        preferred_element_type=jnp.float32)
    o_ref[...] = (p + db).reshape(1, 1, _ROWS_PER_BLK)


def _node_scores(task_repr, Wt, b2):
    return pl.pallas_call(
        _matvec_body,
        grid=(_D_ROWS,),
        in_specs=[
            pl.BlockSpec((_ROWS_PER_BLK, D_FEAT), lambda i: (i, 0)),
            pl.BlockSpec((2, D_FEAT), lambda i: (0, 0)),
            pl.BlockSpec((1, 2), lambda i: (0, 0)),
        ],
        out_specs=pl.BlockSpec((1, 1, _ROWS_PER_BLK), lambda i: (i, 0, 0)),
        out_shape=jax.ShapeDtypeStruct(
            (_D_ROWS, 1, _ROWS_PER_BLK), jnp.float32),
    )(task_repr, Wt, b2)


_sc_mesh = plsc.VectorSubcoreMesh(core_axis_name="c", subcore_axis_name="s")


@functools.partial(
    pl.kernel,
    out_type=(
        jax.ShapeDtypeStruct((N_EDGES,), jnp.float32),
        jax.ShapeDtypeStruct((N_EDGES,), jnp.float32),
    ),
    mesh=_sc_mesh,
    compiler_params=pltpu.CompilerParams(needs_layout_passes=False),
    scratch_types=[
        pltpu.VMEM((_D_ROWS * _ROWS_PER_BLK,), jnp.float32),
        pltpu.VMEM((2, _EPW_PAD), jnp.int32),
        pltpu.VMEM((_EPW_PAD,), jnp.float32),
        pltpu.VMEM((_EPW_PAD,), jnp.float32),
        pltpu.SemaphoreType.DMA,
        pltpu.SemaphoreType.DMA,
        pltpu.SemaphoreType.DMA,
        pltpu.SemaphoreType.DMA,
    ],
)
def _edge_softmax(d_hbm, edge_hbm, ac_hbm, at_hbm,
                  d_v, edge_v, ac_v, at_v, sem0, sem1, sem2, sem3):
    wid = lax.axis_index("s") * _NC + lax.axis_index("c")
    base = wid * _EPW
    cp_ds = [
        pltpu.async_copy(
            d_hbm.at[i, 0], d_v.at[pl.ds(i * _ROWS_PER_BLK, _ROWS_PER_BLK)],
            sem0)
        for i in range(_D_ROWS)
    ]
    cp_e = pltpu.async_copy(
        edge_hbm.at[:, pl.ds(base, _EPW)], edge_v.at[:, pl.ds(0, _EPW)], sem1)
    for cp in cp_ds:
        cp.wait()
    cp_e.wait()

    def lanes(off):
        r = edge_v[0, pl.ds(off, _L)]
        c = edge_v[1, pl.ds(off, _L)]
        s = plsc.load_gather(d_v, [r]) + plsc.load_gather(d_v, [c])
        ac = 1.0 / (1.0 + jnp.exp(-s))
        ac_v[pl.ds(off, _L)] = ac
        at_v[pl.ds(off, _L)] = 1.0 - ac

    @plsc.parallel_loop(0, _EPW, step=_L, unroll=12)
    def _loop(off):
        lanes(off)

    cp_ac = pltpu.async_copy(
        ac_v.at[pl.ds(0, _EPW)], ac_hbm.at[pl.ds(base, _EPW)], sem0)
    cp_at = pltpu.async_copy(
        at_v.at[pl.ds(0, _EPW)], at_hbm.at[pl.ds(base, _EPW)], sem1)

    @pl.when(wid < 2)
    def _extra():
        xbase = _NW * _EPW + wid * _XB
        cp_x = pltpu.async_copy(
            edge_hbm.at[:, pl.ds(xbase, _XB)],
            edge_v.at[:, pl.ds(_EPW, _XB)], sem2)
        cp_x.wait()
        for j in range(_XB // _L):
            lanes(_EPW + j * _L)
        cp_xac = pltpu.async_copy(
            ac_v.at[pl.ds(_EPW, _XB)], ac_hbm.at[pl.ds(xbase, _XB)], sem2)
        cp_xat = pltpu.async_copy(
            at_v.at[pl.ds(_EPW, _XB)], at_hbm.at[pl.ds(xbase, _XB)], sem3)
        cp_xac.wait()
        cp_xat.wait()

    cp_ac.wait()
    cp_at.wait()


def kernel(task_repr, task_edge, W, b):
    d = _node_scores(task_repr, W.T, b.reshape(1, 2))
    a_causual, a_trivial = _edge_softmax(d, task_edge.astype(jnp.int32))
    return (a_causual, a_trivial)


# speedup vs baseline: 1.0038x; 1.0038x over previous
import functools

import jax
import jax.numpy as jnp
from jax import lax
from jax.experimental import pallas as pl
from jax.experimental.pallas import tpu as pltpu
from jax.experimental.pallas import tpu_sc as plsc

N_NODES = 10000
N_EDGES = 160000
D_FEAT = 256

_ROWS_PER_BLK = 2048

_info = plsc.get_sparse_core_info()
_NC, _NS, _L = _info.num_cores, _info.num_subcores, _info.num_lanes
_NW = _NC * _NS
_EPW = 4992
_XB = 128
_EPW_PAD = _EPW + _XB


_D_ROWS = (N_NODES + _ROWS_PER_BLK - 1) // _ROWS_PER_BLK


def _matvec_body(x_ref, wt_ref, b_ref, o_ref):
    wv = wt_ref[0:1, :] - wt_ref[1:2, :]
    db = (b_ref[0, 0] - b_ref[0, 1]) * 0.5
    p = lax.dot_general(
        wv, x_ref[...], (((1,), (1,)), ((), ())),
        preferred_element_type=jnp.float32)
    o_ref[...] = (p + db).reshape(1, 1, _ROWS_PER_BLK)


def _node_scores(task_repr, Wt, b2):
    return pl.pallas_call(
        _matvec_body,
        grid=(_D_ROWS,),
        in_specs=[
            pl.BlockSpec((_ROWS_PER_BLK, D_FEAT), lambda i: (i, 0)),
            pl.BlockSpec((2, D_FEAT), lambda i: (0, 0)),
            pl.BlockSpec((1, 2), lambda i: (0, 0)),
        ],
        out_specs=pl.BlockSpec((1, 1, _ROWS_PER_BLK), lambda i: (i, 0, 0)),
        out_shape=jax.ShapeDtypeStruct(
            (_D_ROWS, 1, _ROWS_PER_BLK), jnp.float32),
    )(task_repr, Wt, b2)


_sc_mesh = plsc.VectorSubcoreMesh(core_axis_name="c", subcore_axis_name="s")


@functools.partial(
    pl.kernel,
    out_type=(
        jax.ShapeDtypeStruct((N_EDGES,), jnp.float32),
        jax.ShapeDtypeStruct((N_EDGES,), jnp.float32),
    ),
    mesh=_sc_mesh,
    compiler_params=pltpu.CompilerParams(needs_layout_passes=False),
    scratch_types=[
        pltpu.VMEM((_D_ROWS * _ROWS_PER_BLK,), jnp.float32),
        pltpu.VMEM((2, _EPW_PAD), jnp.int32),
        pltpu.VMEM((_EPW_PAD,), jnp.float32),
        pltpu.VMEM((_EPW_PAD,), jnp.float32),
        pltpu.SemaphoreType.DMA,
        pltpu.SemaphoreType.DMA,
        pltpu.SemaphoreType.DMA,
        pltpu.SemaphoreType.DMA,
    ],
)
def _edge_softmax(d_hbm, edge_hbm, ac_hbm, at_hbm,
                  d_v, edge_v, ac_v, at_v, sem0, sem1, sem2, sem3):
    wid = lax.axis_index("s") * _NC + lax.axis_index("c")
    base = wid * _EPW
    cp_ds = [
        pltpu.async_copy(
            d_hbm.at[i, 0], d_v.at[pl.ds(i * _ROWS_PER_BLK, _ROWS_PER_BLK)],
            sem0)
        for i in range(_D_ROWS)
    ]
    cp_e = pltpu.async_copy(
        edge_hbm.at[:, pl.ds(base, _EPW)], edge_v.at[:, pl.ds(0, _EPW)], sem1)
    for cp in cp_ds:
        cp.wait()
    cp_e.wait()

    def lanes(off):
        r = edge_v[0, pl.ds(off, _L)]
        c = edge_v[1, pl.ds(off, _L)]
        s = plsc.load_gather(d_v, [r]) + plsc.load_gather(d_v, [c])
        ac = 1.0 / (1.0 + jnp.exp(-s))
        ac_v[pl.ds(off, _L)] = ac
        at_v[pl.ds(off, _L)] = 1.0 - ac

    @plsc.parallel_loop(0, _EPW, step=_L, unroll=12)
    def _loop(off):
        lanes(off)

    cp_ac = pltpu.async_copy(
        ac_v.at[pl.ds(0, _EPW)], ac_hbm.at[pl.ds(base, _EPW)], sem0)
    cp_at = pltpu.async_copy(
        at_v.at[pl.ds(0, _EPW)], at_hbm.at[pl.ds(base, _EPW)], sem1)

    @pl.when(wid < 2)
    def _extra():
        xbase = _NW * _EPW + wid * _XB
        cp_x = pltpu.async_copy(
            edge_hbm.at[:, pl.ds(xbase, _XB)],
            edge_v.at[:, pl.ds(_EPW, _XB)], sem2)
        cp_x.wait()
        for j in range(_XB // _L):
            lanes(_EPW + j * _L)
        cp_xac = pltpu.async_copy(
            ac_v.at[pl.ds(_EPW, _XB)], ac_hbm.at[pl.ds(xbase, _XB)], sem2)
        cp_xat = pltpu.async_copy(
            at_v.at[pl.ds(_EPW, _XB)], at_hbm.at[pl.ds(xbase, _XB)], sem3)
        cp_xac.wait()
        cp_xat.wait()

    cp_ac.wait()
    cp_at.wait()


def kernel(task_repr, task_edge, W, b):
    d = _node_scores(task_repr, W.T, b.reshape(1, 2))
    a_causual, a_trivial = _edge_softmax(d, task_edge.astype(jnp.int32))
    return (a_causual, a_trivial)
